# baseline (speedup 1.0000x reference)
import functools

import jax
import jax.numpy as jnp
from jax import lax
from jax.experimental import pallas as pl
from jax.experimental.pallas import tpu as pltpu
from jax.experimental.pallas import tpu_sc as plsc

_DIM = 64
_PAD = 128
_CHUNK = 128
_K = 2
_NC = 2
_NS = 16
_NW = _NC * _NS


def _embed_body(table_hbm, idx_hbm, out_hbm, idx_v, gbuf, gsem, wsem):
    wid = lax.axis_index("s") * _NC + lax.axis_index("c")
    rows_total = idx_hbm.shape[0]
    per_w = rows_total // _NW
    base = wid * per_w
    seq = out_hbm.shape[1]
    grp = _K * _CHUNK
    n_groups = per_w // _K

    pltpu.sync_copy(idx_hbm.at[pl.ds(base, per_w)], idx_v)

    def fire_gathers(g, s):
        for j in range(_K):
            pltpu.async_copy(
                table_hbm.at[idx_v.at[g * _K + j]],
                gbuf.at[s, pl.ds(j * _CHUNK, _CHUNK)],
                gsem,
            )

    def wait_gathers(s):
        pltpu.make_async_copy(table_hbm.at[pl.ds(0, grp)], gbuf.at[s], gsem).wait()

    def fire_writeback(g, s):
        t0 = (base + g * _K) * _CHUNK
        b = t0 // seq
        s0 = t0 % seq
        pltpu.async_copy(gbuf.at[s], out_hbm.at[b, pl.ds(s0, grp)], wsem)

    def wait_writeback(s):
        pltpu.make_async_copy(gbuf.at[s], out_hbm.at[0, pl.ds(0, grp)], wsem).wait()

    fire_gathers(0, 0)

    def pair_body(i, carry):
        for s in (0, 1):
            g = 2 * i + s

            @pl.when(g + 1 < n_groups)
            def _fire_next():
                if s == 1:
                    wait_writeback(1 - s)
                else:

                    @pl.when(g >= 1)
                    def _():
                        wait_writeback(1 - s)

                fire_gathers(g + 1, 1 - s)

            wait_gathers(s)
            fire_writeback(g, s)
        return carry

    lax.fori_loop(0, n_groups // 2, pair_body, 0)
    wait_writeback(0)
    wait_writeback(1)


@functools.partial(jax.jit, static_argnames=("batch", "seq"))
def _gather_rows(table_pad, idx2d, batch, seq):
    mesh = plsc.VectorSubcoreMesh(core_axis_name="c", subcore_axis_name="s")
    n_rows = idx2d.shape[0]
    fn = functools.partial(
        pl.kernel,
        mesh=mesh,
        out_type=jax.ShapeDtypeStruct((batch, seq, _PAD), jnp.float32),
        scratch_types=[
            pltpu.VMEM((n_rows // _NW, _CHUNK), jnp.int32),
            pltpu.VMEM((2, _K * _CHUNK, _PAD), jnp.float32),
            pltpu.SemaphoreType.DMA,
            pltpu.SemaphoreType.DMA,
        ],
        compiler_params=pltpu.CompilerParams(use_tc_tiling_on_sc=False),
    )(_embed_body)
    return fn(table_pad, idx2d)


def kernel(input_ids, attention_mask, table):
    b, s = input_ids.shape
    n_rows = (b * s) // _CHUNK
    idx2d = input_ids.reshape(n_rows, _CHUNK)
    half = table.shape[0] // 2
    table_pad = jnp.concatenate(
        [
            jnp.pad(table[:half], ((0, 0), (0, _PAD - _DIM))),
            jnp.pad(table[half:], ((0, 0), (0, _PAD - _DIM))),
        ],
        axis=0,
    )
    out128 = _gather_rows(table_pad, idx2d, b, s)
    token_embeddings = out128[:, :, :_DIM]
    return (input_ids, token_embeddings, attention_mask)


# speedup vs baseline: 1.4580x; 1.4580x over previous
import functools

import jax
import jax.numpy as jnp
from jax import lax
from jax.experimental import pallas as pl
from jax.experimental.pallas import tpu as pltpu
from jax.experimental.pallas import tpu_sc as plsc

_DIM = 64
_PAD = 128
_CHUNK = 128
_K = 2
_NC = 2
_NS = 16
_NW = _NC * _NS


def _embed_body(table_hbm, idx_hbm, out_hbm, idx_v, gbuf, gsem, wsem):
    wid = lax.axis_index("s") * _NC + lax.axis_index("c")
    rows_total = idx_hbm.shape[0]
    per_w = rows_total // _NW
    base = wid * per_w
    seq = out_hbm.shape[1]
    grp = _K * _CHUNK
    n_groups = per_w // _K

    pltpu.sync_copy(idx_hbm.at[pl.ds(base, per_w)], idx_v)

    def fire_gathers(g, s):
        for j in range(_K):
            pltpu.async_copy(
                table_hbm.at[idx_v.at[g * _K + j]],
                gbuf.at[s, pl.ds(j * _CHUNK, _CHUNK)],
                gsem,
            )

    def wait_gathers(s):
        pltpu.make_async_copy(table_hbm.at[pl.ds(0, grp)], gbuf.at[s], gsem).wait()

    def fire_writeback(g, s):
        t0 = (base + g * _K) * _CHUNK
        b = t0 // seq
        s0 = t0 % seq
        pltpu.async_copy(
            gbuf.at[s], out_hbm.at[b, pl.ds(s0, grp), pl.ds(0, _DIM)], wsem
        )

    def wait_writeback(s):
        pltpu.make_async_copy(
            gbuf.at[s], out_hbm.at[0, pl.ds(0, grp), pl.ds(0, _DIM)], wsem
        ).wait()

    fire_gathers(0, 0)

    def pair_body(i, carry):
        for s in (0, 1):
            g = 2 * i + s

            @pl.when(g + 1 < n_groups)
            def _fire_next():
                if s == 1:
                    wait_writeback(1 - s)
                else:

                    @pl.when(g >= 1)
                    def _():
                        wait_writeback(1 - s)

                fire_gathers(g + 1, 1 - s)

            wait_gathers(s)
            fire_writeback(g, s)
        return carry

    lax.fori_loop(0, n_groups // 2, pair_body, 0)
    wait_writeback(0)
    wait_writeback(1)


@functools.partial(jax.jit, static_argnames=("batch", "seq"))
def _gather_rows(table_pad, idx2d, batch, seq):
    mesh = plsc.VectorSubcoreMesh(core_axis_name="c", subcore_axis_name="s")
    n_rows = idx2d.shape[0]
    fn = functools.partial(
        pl.kernel,
        mesh=mesh,
        out_type=jax.ShapeDtypeStruct((batch, seq, _PAD), jnp.float32),
        scratch_types=[
            pltpu.VMEM((n_rows // _NW, _CHUNK), jnp.int32),
            pltpu.VMEM((2, _K * _CHUNK, _DIM), jnp.float32),
            pltpu.SemaphoreType.DMA,
            pltpu.SemaphoreType.DMA,
        ],
        compiler_params=pltpu.CompilerParams(use_tc_tiling_on_sc=False),
    )(_embed_body)
    return fn(table_pad, idx2d)


def kernel(input_ids, attention_mask, table):
    b, s = input_ids.shape
    n_rows = (b * s) // _CHUNK
    idx2d = input_ids.reshape(n_rows, _CHUNK)
    out128 = _gather_rows(table, idx2d, b, s)
    token_embeddings = out128[:, :, :_DIM]
    return (input_ids, token_embeddings, attention_mask)


# speedup vs baseline: 1.4596x; 1.0011x over previous
import functools

import jax
import jax.numpy as jnp
from jax import lax
from jax.experimental import pallas as pl
from jax.experimental.pallas import tpu as pltpu
from jax.experimental.pallas import tpu_sc as plsc

_DIM = 64
_PAD = 128
_CHUNK = 128
_K = 4
_NC = 2
_NS = 16
_NW = _NC * _NS


def _embed_body(table_hbm, idx_hbm, out_hbm, idx_v, gbuf, gsem, wsem):
    wid = lax.axis_index("s") * _NC + lax.axis_index("c")
    rows_total = idx_hbm.shape[0]
    per_w = rows_total // _NW
    base = wid * per_w
    seq = out_hbm.shape[1]
    grp = _K * _CHUNK
    n_groups = per_w // _K

    pltpu.sync_copy(idx_hbm.at[pl.ds(base, per_w)], idx_v)

    def fire_gathers(g, s):
        for j in range(_K):
            pltpu.async_copy(
                table_hbm.at[idx_v.at[g * _K + j]],
                gbuf.at[s, pl.ds(j * _CHUNK, _CHUNK)],
                gsem,
            )

    def wait_gathers(s):
        pltpu.make_async_copy(table_hbm.at[pl.ds(0, grp)], gbuf.at[s], gsem).wait()

    def fire_writeback(g, s):
        t0 = (base + g * _K) * _CHUNK
        b = t0 // seq
        s0 = t0 % seq
        pltpu.async_copy(
            gbuf.at[s], out_hbm.at[b, pl.ds(s0, grp), pl.ds(0, _DIM)], wsem
        )

    def wait_writeback(s):
        pltpu.make_async_copy(
            gbuf.at[s], out_hbm.at[0, pl.ds(0, grp), pl.ds(0, _DIM)], wsem
        ).wait()

    fire_gathers(0, 0)

    def pair_body(i, carry):
        for s in (0, 1):
            g = 2 * i + s

            @pl.when(g + 1 < n_groups)
            def _fire_next():
                if s == 1:
                    wait_writeback(1 - s)
                else:

                    @pl.when(g >= 1)
                    def _():
                        wait_writeback(1 - s)

                fire_gathers(g + 1, 1 - s)

            wait_gathers(s)
            fire_writeback(g, s)
        return carry

    lax.fori_loop(0, n_groups // 2, pair_body, 0)
    wait_writeback(0)
    wait_writeback(1)


@functools.partial(jax.jit, static_argnames=("batch", "seq"))
def _gather_rows(table_pad, idx2d, batch, seq):
    mesh = plsc.VectorSubcoreMesh(core_axis_name="c", subcore_axis_name="s")
    n_rows = idx2d.shape[0]
    fn = functools.partial(
        pl.kernel,
        mesh=mesh,
        out_type=jax.ShapeDtypeStruct((batch, seq, _PAD), jnp.float32),
        scratch_types=[
            pltpu.VMEM((n_rows // _NW, _CHUNK), jnp.int32),
            pltpu.VMEM((2, _K * _CHUNK, _DIM), jnp.float32),
            pltpu.SemaphoreType.DMA,
            pltpu.SemaphoreType.DMA,
        ],
        compiler_params=pltpu.CompilerParams(use_tc_tiling_on_sc=False),
    )(_embed_body)
    return fn(table_pad, idx2d)


def kernel(input_ids, attention_mask, table):
    b, s = input_ids.shape
    n_rows = (b * s) // _CHUNK
    idx2d = input_ids.reshape(n_rows, _CHUNK)
    out128 = _gather_rows(table, idx2d, b, s)
    token_embeddings = out128[:, :, :_DIM]
    return (input_ids, token_embeddings, attention_mask)
